# Initial kernel scaffold; baseline (speedup 1.0000x reference)
import jax
import jax.numpy as jnp
from jax.experimental import pallas as pl


def kernel(edges, features, W0, b0, W1, b1, W2, b2):
    raise NotImplementedError("write your pallas kernel here")


# speedup vs baseline: 7.9583x; 7.9583x over previous
import functools

import jax
import jax.numpy as jnp
from jax import lax
from jax.experimental import pallas as pl
from jax.experimental.pallas import tpu as pltpu
from jax.experimental.pallas import tpu_sc as plsc

_N = 10000
_NP = 10240
_E = 320000
_EPP = 331776
_RW = 128
_ER = _EPP // _RW
_NSUB = 16
_NCORE = 2
_NODES_PER_TILE = _NP // _NSUB
_AGG_ROWS = _ER // _NSUB
_PREP_ROWS = _ER // (_NSUB * _NCORE)
_NBUF = 3
_BR = 1024


def _sc_mesh():
    return plsc.VectorSubcoreMesh(core_axis_name="c", subcore_axis_name="s")


def _prep_body(dst2d, degp, didx, ones_v, zero_v, acc, sem):
    c = lax.axis_index("c")
    s = lax.axis_index("s")
    one16 = jnp.full((16,), 1.0, jnp.float32)
    zero16 = jnp.zeros((16,), jnp.float32)

    def initbufs(i, carry):
        ones_v[i, :] = one16
        zero_v[i, :] = zero16
        return carry

    lax.fori_loop(0, _RW, initbufs, 0)

    r0 = s * _NODES_PER_TILE
    for k in range(_NODES_PER_TILE // _RW):
        pltpu.sync_copy(zero_v, acc.at[pl.ds(r0 + _RW * k, _RW)])

    t = s * _NCORE + c
    pltpu.sync_copy(dst2d.at[t], didx)
    plsc.subcore_barrier()

    def fire(j, carry):
        pltpu.sync_copy(ones_v, acc.at[didx.at[j]], add=True)
        return carry

    lax.fori_loop(0, _PREP_ROWS, fire, 0)
    plsc.subcore_barrier()

    for k in range(_NODES_PER_TILE // _RW):
        pltpu.sync_copy(acc.at[pl.ds(r0 + _RW * k, _RW)], ones_v)
        pltpu.sync_copy(ones_v, degp.at[pl.ds(c * _NP + r0 + _RW * k, _RW)])


_sc_prep = functools.partial(
    pl.kernel,
    out_type=jax.ShapeDtypeStruct((_NCORE * _NP, 16), jnp.float32),
    mesh=_sc_mesh(),
    scratch_types=[
        pltpu.VMEM((_PREP_ROWS, _RW), jnp.int32),
        pltpu.VMEM((_RW, 16), jnp.float32),
        pltpu.VMEM((_RW, 16), jnp.float32),
        pltpu.VMEM_SHARED((_NP, 16), jnp.float32),
        pltpu.SemaphoreType.DMA,
    ],
    compiler_params=pltpu.CompilerParams(use_tc_tiling_on_sc=False),
    name="scdeg",
)(_prep_body)


_WC = 64


def _agg_body(src2d, dst2d, xsflat, harr, out, sbuf, dbuf, rb0, rb1, rb2,
              hbuf, acc, g0, g1, g2):
    c = lax.axis_index("c")
    s = lax.axis_index("s")
    rows = [rb0, rb1, rb2]
    gs = [g0, g1, g2]
    zero16 = jnp.zeros((16,), jnp.float32)

    pltpu.sync_copy(harr, hbuf)

    def zb(i, carry):
        for k in range(_WC // 16):
            rb0[i, pl.ds(16 * k, 16)] = zero16
        return carry

    lax.fori_loop(0, _RW, zb, 0)
    nr0 = s * _NODES_PER_TILE
    for k in range(_NODES_PER_TILE // _RW):
        pltpu.sync_copy(rb0, acc.at[pl.ds(nr0 + _RW * k, _RW)])

    pltpu.sync_copy(src2d.at[s], sbuf)
    pltpu.sync_copy(dst2d.at[s], dbuf)

    qvec = hbuf[...] * 2 + c

    def fix(j, carry):
        for k in range(_RW // 16):
            v = sbuf[j, pl.ds(16 * k, 16)]
            sbuf[j, pl.ds(16 * k, 16)] = v * 4 + qvec
        return carry

    lax.fori_loop(0, _AGG_ROWS, fix, 0)
    plsc.subcore_barrier()

    def chunk(j, carry):
        pltpu.sync_copy(xsflat.at[sbuf.at[j]], rows[0])
        pltpu.sync_copy(rows[0], acc.at[dbuf.at[j]], add=True)
        return carry

    lax.fori_loop(0, _AGG_ROWS, chunk, 0)
    plsc.subcore_barrier()

    for k in range(_NODES_PER_TILE // _RW):
        pltpu.sync_copy(acc.at[pl.ds(nr0 + _RW * k, _RW)], rb0)
        pltpu.sync_copy(
            rb0, out.at[pl.ds(c * _NP + nr0 + _RW * k, _RW)])


_agg64 = functools.partial(
    pl.kernel,
    out_type=jax.ShapeDtypeStruct((_NCORE * _NP, _WC), jnp.float32),
    mesh=_sc_mesh(),
    scratch_types=[
        pltpu.VMEM((_AGG_ROWS, _RW), jnp.int32),
        pltpu.VMEM((_AGG_ROWS, _RW), jnp.int32),
        pltpu.VMEM((_RW, _WC), jnp.float32),
        pltpu.VMEM((_RW, _WC), jnp.float32),
        pltpu.VMEM((_RW, _WC), jnp.float32),
        pltpu.VMEM((16,), jnp.int32),
        pltpu.VMEM_SHARED((_NP, _WC), jnp.float32),
        pltpu.SemaphoreType.DMA,
        pltpu.SemaphoreType.DMA,
        pltpu.SemaphoreType.DMA,
    ],
    compiler_params=pltpu.CompilerParams(use_tc_tiling_on_sc=False),
    name="scagg",
)(_agg_body)


def _dinv_of(dp):
    deg = jnp.maximum(dp[0, :, 0:1] + dp[1, :, 0:1], 1.0)
    y = lax.rsqrt(deg)
    return y * (1.5 - 0.5 * deg * y * y)


def _tc0_body(f_ref, w_ref, dp_ref, o_ref):
    dinv = _dinv_of(dp_ref[...])
    o_ref[...] = jnp.dot(
        f_ref[...], w_ref[...], preferred_element_type=jnp.float32,
        precision=lax.Precision.HIGHEST) * dinv


def _tcmid_body(a_ref, dp_ref, b_ref, w_ref, o_ref):
    dinv = _dinv_of(dp_ref[...])
    h = jnp.maximum(a_ref[...] * dinv + b_ref[...], 0.0)
    o_ref[...] = jnp.dot(
        h, w_ref[...], preferred_element_type=jnp.float32,
        precision=lax.Precision.HIGHEST) * dinv


def _tcfinal_body(a_ref, dp_ref, b_ref, o_ref):
    dinv = _dinv_of(dp_ref[...])
    z = a_ref[...] * dinv + b_ref[...]
    m = jnp.max(z, axis=1, keepdims=True)
    e = jnp.exp(z - m)
    ssum = jnp.sum(e, axis=1, keepdims=True)
    o_ref[...] = z - m - jnp.log(ssum)


def _tc0(fpad, w0, degp2):
    return pl.pallas_call(
        _tc0_body,
        grid=(_NP // _BR,),
        in_specs=[
            pl.BlockSpec((_BR, 128), lambda i: (i, 0)),
            pl.BlockSpec((128, 256), lambda i: (0, 0)),
            pl.BlockSpec((2, _BR, 16), lambda i: (0, i, 0)),
        ],
        out_specs=pl.BlockSpec((_BR, 256), lambda i: (i, 0)),
        out_shape=jax.ShapeDtypeStruct((_NP, 256), jnp.float32),
    )(fpad, w0, degp2)


def _tcmid(acc, degp2, b, w, dout):
    return pl.pallas_call(
        _tcmid_body,
        grid=(_NP // _BR,),
        in_specs=[
            pl.BlockSpec((_BR, 256), lambda i: (i, 0)),
            pl.BlockSpec((2, _BR, 16), lambda i: (0, i, 0)),
            pl.BlockSpec((1, 256), lambda i: (0, 0)),
            pl.BlockSpec((256, dout), lambda i: (0, 0)),
        ],
        out_specs=pl.BlockSpec((_BR, dout), lambda i: (i, 0)),
        out_shape=jax.ShapeDtypeStruct((_NP, dout), jnp.float32),
    )(acc, degp2, b, w)


def _tcfinal(acc, degp2, b):
    return pl.pallas_call(
        _tcfinal_body,
        grid=(_NP // _BR,),
        in_specs=[
            pl.BlockSpec((_BR, 64), lambda i: (i, 0)),
            pl.BlockSpec((2, _BR, 16), lambda i: (0, i, 0)),
            pl.BlockSpec((1, 64), lambda i: (0, 0)),
        ],
        out_specs=pl.BlockSpec((_BR, 64), lambda i: (i, 0)),
        out_shape=jax.ShapeDtypeStruct((_NP, 64), jnp.float32),
    )(acc, degp2, b)


def _cat_halves(a, wc):
    return jnp.concatenate([a[:_NP], a[_NP:]], axis=1)


@jax.jit
def kernel(edges, features, W0, b0, W1, b1, W2, b2):
    src = edges[0].astype(jnp.int32)
    dst = edges[1].astype(jnp.int32)
    loop = jnp.arange(_N, dtype=jnp.int32)
    padv = jnp.full((_EPP - _E - _N,), _N, jnp.int32)
    src_f = jnp.concatenate([src, loop, padv])
    dst_f = jnp.concatenate([dst, loop, padv])
    src_a = src_f.reshape(_NSUB, _AGG_ROWS, _RW)
    dst_a = dst_f.reshape(_NSUB, _AGG_ROWS, _RW)
    dst_p = dst_f.reshape(_NSUB * _NCORE, _PREP_ROWS, _RW)
    fpad = jnp.zeros((_NP, 128), jnp.float32).at[:_N].set(features)

    degp = _sc_prep(dst_p)
    degp2 = degp.reshape(2, _NP, 16)

    xs0 = _tc0(fpad, W0, degp2)

    ws = jnp.stack([W1, jnp.pad(W2, ((0, 0), (0, 256 - 64)))])
    bs = jnp.stack([b0, b1]).reshape(2, 1, 256)

    def step(j, carry):
        xs, accf = carry
        k = j >> 1
        h = j & 1
        harr = jnp.full((16,), h, jnp.int32)
        blk = _cat_halves(
            _agg64(src_a, dst_a, xs.reshape(4 * _NP, _WC), harr), _WC)
        accf = lax.dynamic_update_slice(accf, blk, (0, 128 * h))

        def do_mm(args):
            a, kk = args
            w = lax.dynamic_index_in_dim(ws, kk, keepdims=False)
            b = lax.dynamic_index_in_dim(bs, kk, keepdims=False)
            return _tcmid(a, degp2, b, w, 256)

        xs = lax.cond(h == 1, do_mm, lambda args: xs, (accf, k))
        return xs, accf

    nsteps = lax.optimization_barrier(jnp.int32(5))
    _, accf = lax.fori_loop(
        0, nsteps, step, (xs0, jnp.zeros((_NP, 256), jnp.float32)))
    out = _tcfinal(accf[:, :64], degp2, b2.reshape(1, -1))
    return out[:_N]


# speedup vs baseline: 9.5075x; 1.1947x over previous
import functools

import jax
import jax.numpy as jnp
from jax import lax
from jax.experimental import pallas as pl
from jax.experimental.pallas import tpu as pltpu
from jax.experimental.pallas import tpu_sc as plsc

_N = 10000
_NP = 10240
_E = 320000
_EPP = 331776
_RW = 128
_ER = _EPP // _RW
_NSUB = 16
_NCORE = 2
_NODES_PER_TILE = _NP // _NSUB
_AGG_ROWS = _ER // _NSUB
_PREP_ROWS = _ER // (_NSUB * _NCORE)
_NBUF = 3
_BR = 1024


def _sc_mesh():
    return plsc.VectorSubcoreMesh(core_axis_name="c", subcore_axis_name="s")


def _prep_body(dst2d, degp, didx, ones_v, zero_v, acc, sem):
    c = lax.axis_index("c")
    s = lax.axis_index("s")
    one16 = jnp.full((16,), 1.0, jnp.float32)
    zero16 = jnp.zeros((16,), jnp.float32)

    def initbufs(i, carry):
        ones_v[i, :] = one16
        zero_v[i, :] = zero16
        return carry

    lax.fori_loop(0, _RW, initbufs, 0)

    r0 = s * _NODES_PER_TILE
    for k in range(_NODES_PER_TILE // _RW):
        pltpu.sync_copy(zero_v, acc.at[pl.ds(r0 + _RW * k, _RW)])

    t = s * _NCORE + c
    pltpu.sync_copy(dst2d.at[t], didx)
    plsc.subcore_barrier()

    def fire(j, carry):
        pltpu.sync_copy(ones_v, acc.at[didx.at[j]], add=True)
        return carry

    lax.fori_loop(0, _PREP_ROWS, fire, 0)
    plsc.subcore_barrier()

    for k in range(_NODES_PER_TILE // _RW):
        pltpu.sync_copy(acc.at[pl.ds(r0 + _RW * k, _RW)], ones_v)
        pltpu.sync_copy(ones_v, degp.at[pl.ds(c * _NP + r0 + _RW * k, _RW)])


_sc_prep = functools.partial(
    pl.kernel,
    out_type=jax.ShapeDtypeStruct((_NCORE * _NP, 16), jnp.float32),
    mesh=_sc_mesh(),
    scratch_types=[
        pltpu.VMEM((_PREP_ROWS, _RW), jnp.int32),
        pltpu.VMEM((_RW, 16), jnp.float32),
        pltpu.VMEM((_RW, 16), jnp.float32),
        pltpu.VMEM_SHARED((_NP, 16), jnp.float32),
        pltpu.SemaphoreType.DMA,
    ],
    compiler_params=pltpu.CompilerParams(use_tc_tiling_on_sc=False),
    name="scdeg",
)(_prep_body)


_WC = 64


def _agg_body(src2d, dst2d, xsflat, harr, out, sbuf, dbuf, rb0, rb1, rb2,
              hbuf, acc, g0, g1, g2):
    c = lax.axis_index("c")
    s = lax.axis_index("s")
    rows = [rb0, rb1, rb2]
    gs = [g0, g1, g2]
    zero16 = jnp.zeros((16,), jnp.float32)

    pltpu.sync_copy(harr, hbuf)

    def zb(i, carry):
        for k in range(_WC // 16):
            rb0[i, pl.ds(16 * k, 16)] = zero16
        return carry

    lax.fori_loop(0, _RW, zb, 0)
    nr0 = s * _NODES_PER_TILE
    for k in range(_NODES_PER_TILE // _RW):
        pltpu.sync_copy(rb0, acc.at[pl.ds(nr0 + _RW * k, _RW)])

    pltpu.sync_copy(src2d.at[s], sbuf)
    pltpu.sync_copy(dst2d.at[s], dbuf)

    qvec = hbuf[...] * 2 + c

    def fix(j, carry):
        for k in range(_RW // 16):
            v = sbuf[j, pl.ds(16 * k, 16)]
            sbuf[j, pl.ds(16 * k, 16)] = v * 4 + qvec
        return carry

    lax.fori_loop(0, _AGG_ROWS, fix, 0)
    plsc.subcore_barrier()

    def group(g, carry):
        j0 = g * _NBUF
        descs = [
            pltpu.async_copy(xsflat.at[sbuf.at[j0 + u]], rows[u], gs[u])
            for u in range(_NBUF)
        ]
        for u in range(_NBUF):
            descs[u].wait()
            pltpu.sync_copy(rows[u], acc.at[dbuf.at[j0 + u]], add=True)
        return carry

    lax.fori_loop(0, _AGG_ROWS // _NBUF, group, 0)
    plsc.subcore_barrier()

    for k in range(_NODES_PER_TILE // _RW):
        pltpu.sync_copy(acc.at[pl.ds(nr0 + _RW * k, _RW)], rb0)
        pltpu.sync_copy(
            rb0, out.at[pl.ds(c * _NP + nr0 + _RW * k, _RW)])


_agg64 = functools.partial(
    pl.kernel,
    out_type=jax.ShapeDtypeStruct((_NCORE * _NP, _WC), jnp.float32),
    mesh=_sc_mesh(),
    scratch_types=[
        pltpu.VMEM((_AGG_ROWS, _RW), jnp.int32),
        pltpu.VMEM((_AGG_ROWS, _RW), jnp.int32),
        pltpu.VMEM((_RW, _WC), jnp.float32),
        pltpu.VMEM((_RW, _WC), jnp.float32),
        pltpu.VMEM((_RW, _WC), jnp.float32),
        pltpu.VMEM((16,), jnp.int32),
        pltpu.VMEM_SHARED((_NP, _WC), jnp.float32),
        pltpu.SemaphoreType.DMA,
        pltpu.SemaphoreType.DMA,
        pltpu.SemaphoreType.DMA,
    ],
    compiler_params=pltpu.CompilerParams(use_tc_tiling_on_sc=False),
    name="scagg",
)(_agg_body)


def _dinv_of(dp):
    deg = jnp.maximum(dp[0, :, 0:1] + dp[1, :, 0:1], 1.0)
    y = lax.rsqrt(deg)
    return y * (1.5 - 0.5 * deg * y * y)


def _tc0_body(f_ref, w_ref, dp_ref, o_ref):
    dinv = _dinv_of(dp_ref[...])
    o_ref[...] = jnp.dot(
        f_ref[...], w_ref[...], preferred_element_type=jnp.float32,
        precision=lax.Precision.HIGHEST) * dinv


def _tcmid_body(a_ref, dp_ref, b_ref, w_ref, o_ref):
    dinv = _dinv_of(dp_ref[...])
    h = jnp.maximum(a_ref[...] * dinv + b_ref[...], 0.0)
    o_ref[...] = jnp.dot(
        h, w_ref[...], preferred_element_type=jnp.float32,
        precision=lax.Precision.HIGHEST) * dinv


def _tcfinal_body(a_ref, dp_ref, b_ref, o_ref):
    dinv = _dinv_of(dp_ref[...])
    z = a_ref[...] * dinv + b_ref[...]
    m = jnp.max(z, axis=1, keepdims=True)
    e = jnp.exp(z - m)
    ssum = jnp.sum(e, axis=1, keepdims=True)
    o_ref[...] = z - m - jnp.log(ssum)


def _tc0(fpad, w0, degp2):
    return pl.pallas_call(
        _tc0_body,
        grid=(_NP // _BR,),
        in_specs=[
            pl.BlockSpec((_BR, 128), lambda i: (i, 0)),
            pl.BlockSpec((128, 256), lambda i: (0, 0)),
            pl.BlockSpec((2, _BR, 16), lambda i: (0, i, 0)),
        ],
        out_specs=pl.BlockSpec((_BR, 256), lambda i: (i, 0)),
        out_shape=jax.ShapeDtypeStruct((_NP, 256), jnp.float32),
    )(fpad, w0, degp2)


def _tcmid(acc, degp2, b, w, dout):
    return pl.pallas_call(
        _tcmid_body,
        grid=(_NP // _BR,),
        in_specs=[
            pl.BlockSpec((_BR, 256), lambda i: (i, 0)),
            pl.BlockSpec((2, _BR, 16), lambda i: (0, i, 0)),
            pl.BlockSpec((1, 256), lambda i: (0, 0)),
            pl.BlockSpec((256, dout), lambda i: (0, 0)),
        ],
        out_specs=pl.BlockSpec((_BR, dout), lambda i: (i, 0)),
        out_shape=jax.ShapeDtypeStruct((_NP, dout), jnp.float32),
    )(acc, degp2, b, w)


def _tcfinal(acc, degp2, b):
    return pl.pallas_call(
        _tcfinal_body,
        grid=(_NP // _BR,),
        in_specs=[
            pl.BlockSpec((_BR, 64), lambda i: (i, 0)),
            pl.BlockSpec((2, _BR, 16), lambda i: (0, i, 0)),
            pl.BlockSpec((1, 64), lambda i: (0, 0)),
        ],
        out_specs=pl.BlockSpec((_BR, 64), lambda i: (i, 0)),
        out_shape=jax.ShapeDtypeStruct((_NP, 64), jnp.float32),
    )(acc, degp2, b)


def _cat_halves(a, wc):
    return jnp.concatenate([a[:_NP], a[_NP:]], axis=1)


@jax.jit
def kernel(edges, features, W0, b0, W1, b1, W2, b2):
    src = edges[0].astype(jnp.int32)
    dst = edges[1].astype(jnp.int32)
    loop = jnp.arange(_N, dtype=jnp.int32)
    padv = jnp.full((_EPP - _E - _N,), _N, jnp.int32)
    src_f = jnp.concatenate([src, loop, padv])
    dst_f = jnp.concatenate([dst, loop, padv])
    src_a = src_f.reshape(_NSUB, _AGG_ROWS, _RW)
    dst_a = dst_f.reshape(_NSUB, _AGG_ROWS, _RW)
    dst_p = dst_f.reshape(_NSUB * _NCORE, _PREP_ROWS, _RW)
    fpad = jnp.zeros((_NP, 128), jnp.float32).at[:_N].set(features)

    degp = _sc_prep(dst_p)
    degp2 = degp.reshape(2, _NP, 16)

    xs0 = _tc0(fpad, W0, degp2)

    ws = jnp.stack([W1, jnp.pad(W2, ((0, 0), (0, 256 - 64)))])
    bs = jnp.stack([b0, b1]).reshape(2, 1, 256)

    def step(j, carry):
        xs, accf = carry
        k = j >> 1
        h = j & 1
        harr = jnp.full((16,), h, jnp.int32)
        blk = _cat_halves(
            _agg64(src_a, dst_a, xs.reshape(4 * _NP, _WC), harr), _WC)
        accf = lax.dynamic_update_slice(accf, blk, (0, 128 * h))

        def do_mm(args):
            a, kk = args
            w = lax.dynamic_index_in_dim(ws, kk, keepdims=False)
            b = lax.dynamic_index_in_dim(bs, kk, keepdims=False)
            return _tcmid(a, degp2, b, w, 256)

        xs = lax.cond(h == 1, do_mm, lambda args: xs, (accf, k))
        return xs, accf

    nsteps = lax.optimization_barrier(jnp.int32(5))
    _, accf = lax.fori_loop(
        0, nsteps, step, (xs0, jnp.zeros((_NP, 256), jnp.float32)))
    out = _tcfinal(accf[:, :64], degp2, b2.reshape(1, -1))
    return out[:_N]
